# Initial kernel scaffold; baseline (speedup 1.0000x reference)
import functools
import jax
import jax.numpy as jnp
from jax import lax
from jax.experimental import pallas as pl
from jax.experimental.pallas import tpu as pltpu
from jax.experimental.pallas import tpu_sc as plsc

N = 10000
E = 320000
D = 128

NC = 2
NS = 16
NW = NC * NS
EPW = E // NW
WIN = 400
NWIN = EPW // WIN

N_PAD = 10240
RPT = N_PAD // NS
HK = 16

_sc_mesh = plsc.VectorSubcoreMesh(core_axis_name="c", subcore_axis_name="s")


def _deg_body(src_hbm, dst_hbm, ones_hbm, zeros_hbm,
              deg_out_hbm, deg_in_hbm,
              idx_v, ones_v, acc_o, acc_i, sem):
    c = lax.axis_index("c")
    s = lax.axis_index("s")
    base = (c * NS + s) * EPW

    row0 = s * RPT
    pltpu.sync_copy(zeros_hbm, acc_o.at[pl.ds(row0, RPT)])
    pltpu.sync_copy(zeros_hbm, acc_i.at[pl.ds(row0, RPT)])
    pltpu.sync_copy(ones_hbm, ones_v)
    plsc.subcore_barrier()

    @pl.loop(0, NWIN)
    def _win(g):
        off = pl.multiple_of(base + g * WIN, WIN)
        pltpu.sync_copy(src_hbm.at[pl.ds(off, WIN)], idx_v)
        pltpu.sync_copy(ones_v, acc_o.at[idx_v], add=True)
        pltpu.sync_copy(dst_hbm.at[pl.ds(off, WIN)], idx_v)
        pltpu.sync_copy(ones_v, acc_i.at[idx_v], add=True)

    plsc.subcore_barrier()
    pltpu.sync_copy(acc_o.at[pl.ds(row0, RPT)], deg_out_hbm.at[c, pl.ds(row0, RPT)])
    pltpu.sync_copy(acc_i.at[pl.ds(row0, RPT)], deg_in_hbm.at[c, pl.ds(row0, RPT)])


_deg_kernel = functools.partial(
    pl.kernel,
    out_type=(jax.ShapeDtypeStruct((NC, N_PAD, HK), jnp.float32),
              jax.ShapeDtypeStruct((NC, N_PAD, HK), jnp.float32)),
    mesh=_sc_mesh,
    scratch_types=[
        pltpu.VMEM((WIN,), jnp.int32),
        pltpu.VMEM((WIN, HK), jnp.float32),
        pltpu.VMEM_SHARED((N_PAD, HK), jnp.float32),
        pltpu.VMEM_SHARED((N_PAD, HK), jnp.float32),
        pltpu.SemaphoreType.DMA,
    ],
)(_deg_body)


def _spmm_body(t_hbm, src_hbm, dst_hbm, zeros_hbm, out_hbm,
               sidx, didx, rows, acc, sem):
    c = lax.axis_index("c")
    s = lax.axis_index("s")
    base = (c * NS + s) * EPW

    row0 = s * RPT
    pltpu.sync_copy(zeros_hbm, acc.at[pl.ds(row0, RPT)])
    plsc.subcore_barrier()

    @pl.loop(0, NWIN)
    def _win(g):
        off = pl.multiple_of(base + g * WIN, WIN)
        pltpu.sync_copy(src_hbm.at[pl.ds(off, WIN)], sidx)
        pltpu.sync_copy(dst_hbm.at[pl.ds(off, WIN)], didx)
        pltpu.async_copy(t_hbm.at[sidx], rows, sem).wait()
        pltpu.sync_copy(rows, acc.at[didx], add=True)

    plsc.subcore_barrier()
    pltpu.sync_copy(acc.at[pl.ds(row0, RPT)], out_hbm.at[c, pl.ds(row0, RPT)])


_spmm_kernel = functools.partial(
    pl.kernel,
    out_type=jax.ShapeDtypeStruct((NC, N_PAD, D), jnp.float32),
    mesh=_sc_mesh,
    scratch_types=[
        pltpu.VMEM((WIN,), jnp.int32),
        pltpu.VMEM((WIN,), jnp.int32),
        pltpu.VMEM((WIN, D), jnp.float32),
        pltpu.VMEM_SHARED((N_PAD, D), jnp.float32),
        pltpu.SemaphoreType.DMA,
    ],
)(_spmm_body)


BLK = 1024
GRID = N_PAD // BLK


def _inv_sqrt_deg(deg_p_ref, i):
    deg = deg_p_ref[0, pl.ds(i * BLK, BLK), :] + deg_p_ref[1, pl.ds(i * BLK, BLK), :]
    inv = jnp.where(deg > 0, lax.rsqrt(jnp.maximum(deg, 1.0)), 0.0)
    return inv[:, 0:1]


def _tc_first_body(x_ref, w_ref, deg_o_ref, out_ref):
    i = pl.program_id(0)
    inv_o = _inv_sqrt_deg(deg_o_ref, i)
    t = jnp.dot(x_ref[...], w_ref[...], preferred_element_type=jnp.float32)
    out_ref[...] = t * inv_o


def _tc_mid_body(ap_ref, w_ref, b_ref, deg_i_ref, deg_o_ref, out_ref):
    i = pl.program_id(0)
    inv_i = _inv_sqrt_deg(deg_i_ref, i)
    inv_o = _inv_sqrt_deg(deg_o_ref, i)
    a = ap_ref[0] + ap_ref[1]
    h = jnp.maximum(a * inv_i + b_ref[...], 0.0)
    t = jnp.dot(h, w_ref[...], preferred_element_type=jnp.float32)
    out_ref[...] = t * inv_o


def _tc_last_body(ap_ref, b_ref, deg_i_ref, out_ref):
    i = pl.program_id(0)
    inv_i = _inv_sqrt_deg(deg_i_ref, i)
    a = ap_ref[0] + ap_ref[1]
    out_ref[...] = a * inv_i + b_ref[...]


def _full(shape):
    return pl.BlockSpec(shape, lambda i: tuple(0 for _ in shape))


_row_spec = pl.BlockSpec((BLK, D), lambda i: (i, 0))
_ap_spec = pl.BlockSpec((NC, BLK, D), lambda i: (0, i, 0))
_deg_spec = _full((NC, N_PAD, HK))

_tc_first = pl.pallas_call(
    _tc_first_body,
    grid=(GRID,),
    in_specs=[_row_spec, _full((D, D)), _deg_spec],
    out_specs=_row_spec,
    out_shape=jax.ShapeDtypeStruct((N_PAD, D), jnp.float32),
)

_tc_mid = pl.pallas_call(
    _tc_mid_body,
    grid=(GRID,),
    in_specs=[_ap_spec, _full((D, D)), _full((1, D)), _deg_spec, _deg_spec],
    out_specs=_row_spec,
    out_shape=jax.ShapeDtypeStruct((N_PAD, D), jnp.float32),
)

_tc_last = pl.pallas_call(
    _tc_last_body,
    grid=(GRID,),
    in_specs=[_ap_spec, _full((1, D)), _deg_spec],
    out_specs=_row_spec,
    out_shape=jax.ShapeDtypeStruct((N_PAD, D), jnp.float32),
)


@jax.jit
def kernel(features, edge_index, W0, b0, W1, b1, W2, b2):
    src = edge_index[0]
    dst = edge_index[1]

    ones_h = jnp.ones((WIN, HK), jnp.float32)
    zeros_h = jnp.zeros((RPT, HK), jnp.float32)
    zeros_d = jnp.zeros((RPT, D), jnp.float32)

    deg_out_p, deg_in_p = _deg_kernel(src, dst, ones_h, zeros_h)

    x_pad = jnp.pad(features, ((0, N_PAD - N), (0, 0)))

    t0 = _tc_first(x_pad, W0, deg_out_p)
    a0 = _spmm_kernel(t0, src, dst, zeros_d)
    t1 = _tc_mid(a0, W1, b0.reshape(1, D), deg_in_p, deg_out_p)
    a1 = _spmm_kernel(t1, src, dst, zeros_d)
    t2 = _tc_mid(a1, W2, b1.reshape(1, D), deg_in_p, deg_out_p)
    a2 = _spmm_kernel(t2, src, dst, zeros_d)
    logits = _tc_last(a2, b2.reshape(1, D), deg_in_p)
    return logits[:N]


# speedup vs baseline: 8.0035x; 8.0035x over previous
import functools
import jax
import jax.numpy as jnp
from jax import lax
from jax.experimental import pallas as pl
from jax.experimental.pallas import tpu as pltpu
from jax.experimental.pallas import tpu_sc as plsc

N = 10000
E = 320000
D = 128

NC = 2
NS = 16
NW = NC * NS
EPW = E // NW
WIN = 128
NWIN = EPW // WIN
TAIL = EPW - NWIN * WIN

N_PAD = 10240
RPT = N_PAD // NS
HK = 16
DW = 128

_sc_mesh = plsc.VectorSubcoreMesh(core_axis_name="c", subcore_axis_name="s")


def _deg_body(src_hbm, dst_hbm, ones_o_hbm, ones_i_hbm, zeros_hbm,
              deg_hbm,
              idx_o, idx_i, idx_ot, idx_it, vals_o, vals_i, acc, sem):
    c = lax.axis_index("c")
    s = lax.axis_index("s")
    base = (c * NS + s) * EPW

    row0 = s * RPT
    pltpu.sync_copy(zeros_hbm, acc.at[pl.ds(row0, RPT)])
    pltpu.sync_copy(ones_o_hbm, vals_o)
    pltpu.sync_copy(ones_i_hbm, vals_i)
    plsc.subcore_barrier()

    @pl.loop(0, NWIN)
    def _win(g):
        off = pl.multiple_of(base + g * WIN, 16)
        pltpu.sync_copy(src_hbm.at[pl.ds(off, WIN)], idx_o)
        pltpu.sync_copy(dst_hbm.at[pl.ds(off, WIN)], idx_i)
        pltpu.sync_copy(vals_o, acc.at[idx_o], add=True)
        pltpu.sync_copy(vals_i, acc.at[idx_i], add=True)

    off_t = pl.multiple_of(base + NWIN * WIN, 16)
    pltpu.sync_copy(src_hbm.at[pl.ds(off_t, TAIL)], idx_ot)
    pltpu.sync_copy(dst_hbm.at[pl.ds(off_t, TAIL)], idx_it)
    pltpu.sync_copy(vals_o.at[pl.ds(0, TAIL)], acc.at[idx_ot], add=True)
    pltpu.sync_copy(vals_i.at[pl.ds(0, TAIL)], acc.at[idx_it], add=True)

    plsc.subcore_barrier()
    pltpu.sync_copy(acc.at[pl.ds(row0, RPT)], deg_hbm.at[c, pl.ds(row0, RPT)])


_deg_kernel = functools.partial(
    pl.kernel,
    out_type=jax.ShapeDtypeStruct((NC, N_PAD, DW), jnp.float32),
    mesh=_sc_mesh,
    scratch_types=[
        pltpu.VMEM((WIN,), jnp.int32),
        pltpu.VMEM((WIN,), jnp.int32),
        pltpu.VMEM((TAIL,), jnp.int32),
        pltpu.VMEM((TAIL,), jnp.int32),
        pltpu.VMEM((WIN, DW), jnp.float32),
        pltpu.VMEM((WIN, DW), jnp.float32),
        pltpu.VMEM_SHARED((N_PAD, DW), jnp.float32),
        pltpu.SemaphoreType.DMA,
    ],
)(_deg_body)


def _spmm_body(t_hbm, src_hbm, dst_hbm, zeros_hbm, out_hbm,
               sidx, didx, sidx_t, didx_t, rows, rows_t, acc, sem):
    c = lax.axis_index("c")
    s = lax.axis_index("s")
    base = (c * NS + s) * EPW

    row0 = s * RPT
    pltpu.sync_copy(zeros_hbm, acc.at[pl.ds(row0, RPT)])
    plsc.subcore_barrier()

    @pl.loop(0, NWIN)
    def _win(g):
        off = pl.multiple_of(base + g * WIN, 16)
        pltpu.sync_copy(src_hbm.at[pl.ds(off, WIN)], sidx)
        pltpu.sync_copy(dst_hbm.at[pl.ds(off, WIN)], didx)
        pltpu.async_copy(t_hbm.at[sidx], rows, sem).wait()
        pltpu.sync_copy(rows, acc.at[didx], add=True)

    off_t = pl.multiple_of(base + NWIN * WIN, 16)
    pltpu.sync_copy(src_hbm.at[pl.ds(off_t, TAIL)], sidx_t)
    pltpu.sync_copy(dst_hbm.at[pl.ds(off_t, TAIL)], didx_t)
    pltpu.async_copy(t_hbm.at[sidx_t], rows_t, sem).wait()
    pltpu.sync_copy(rows_t, acc.at[didx_t], add=True)

    plsc.subcore_barrier()
    pltpu.sync_copy(acc.at[pl.ds(row0, RPT)], out_hbm.at[c, pl.ds(row0, RPT)])


_spmm_kernel = functools.partial(
    pl.kernel,
    out_type=jax.ShapeDtypeStruct((NC, N_PAD, D), jnp.float32),
    mesh=_sc_mesh,
    scratch_types=[
        pltpu.VMEM((WIN,), jnp.int32),
        pltpu.VMEM((WIN,), jnp.int32),
        pltpu.VMEM((TAIL,), jnp.int32),
        pltpu.VMEM((TAIL,), jnp.int32),
        pltpu.VMEM((WIN, D), jnp.float32),
        pltpu.VMEM((TAIL, D), jnp.float32),
        pltpu.VMEM_SHARED((N_PAD, D), jnp.float32),
        pltpu.SemaphoreType.DMA,
    ],
)(_spmm_body)


BLK = 1024
GRID = N_PAD // BLK


def _inv_sqrt_deg(deg_p_ref, i, col):
    deg = (deg_p_ref[0, pl.ds(i * BLK, BLK), pl.ds(col, HK)]
           + deg_p_ref[1, pl.ds(i * BLK, BLK), pl.ds(col, HK)])
    inv = jnp.where(deg > 0, lax.rsqrt(jnp.maximum(deg, 1.0)), 0.0)
    return inv[:, 0:1]


def _tc_first_body(x_ref, w_ref, deg_ref, out_ref):
    i = pl.program_id(0)
    inv_o = _inv_sqrt_deg(deg_ref, i, 0)
    t = jnp.dot(x_ref[...], w_ref[...], preferred_element_type=jnp.float32)
    out_ref[...] = t * inv_o


def _tc_mid_body(ap_ref, w_ref, b_ref, deg_ref, out_ref):
    i = pl.program_id(0)
    inv_i = _inv_sqrt_deg(deg_ref, i, HK)
    inv_o = _inv_sqrt_deg(deg_ref, i, 0)
    a = ap_ref[0] + ap_ref[1]
    h = jnp.maximum(a * inv_i + b_ref[...], 0.0)
    t = jnp.dot(h, w_ref[...], preferred_element_type=jnp.float32)
    out_ref[...] = t * inv_o


def _tc_last_body(ap_ref, b_ref, deg_ref, out_ref):
    i = pl.program_id(0)
    inv_i = _inv_sqrt_deg(deg_ref, i, HK)
    a = ap_ref[0] + ap_ref[1]
    out_ref[...] = a * inv_i + b_ref[...]


def _full(shape):
    return pl.BlockSpec(shape, lambda i: tuple(0 for _ in shape))


_row_spec = pl.BlockSpec((BLK, D), lambda i: (i, 0))
_ap_spec = pl.BlockSpec((NC, BLK, D), lambda i: (0, i, 0))
_deg_spec = _full((NC, N_PAD, DW))

_tc_first = pl.pallas_call(
    _tc_first_body,
    grid=(GRID,),
    in_specs=[_row_spec, _full((D, D)), _deg_spec],
    out_specs=_row_spec,
    out_shape=jax.ShapeDtypeStruct((N_PAD, D), jnp.float32),
)

_tc_mid = pl.pallas_call(
    _tc_mid_body,
    grid=(GRID,),
    in_specs=[_ap_spec, _full((D, D)), _full((1, D)), _deg_spec],
    out_specs=_row_spec,
    out_shape=jax.ShapeDtypeStruct((N_PAD, D), jnp.float32),
)

_tc_last = pl.pallas_call(
    _tc_last_body,
    grid=(GRID,),
    in_specs=[_ap_spec, _full((1, D)), _deg_spec],
    out_specs=_row_spec,
    out_shape=jax.ShapeDtypeStruct((N_PAD, D), jnp.float32),
)


@jax.jit
def kernel(features, edge_index, W0, b0, W1, b1, W2, b2):
    src = edge_index[0]
    dst = edge_index[1]

    col = jnp.arange(DW)
    ones_o = jnp.where(col < HK, 1.0, 0.0).astype(jnp.float32) * jnp.ones((WIN, 1), jnp.float32)
    ones_i = jnp.where((col >= HK) & (col < 2 * HK), 1.0, 0.0).astype(jnp.float32) * jnp.ones((WIN, 1), jnp.float32)
    zeros_h = jnp.zeros((RPT, DW), jnp.float32)
    zeros_d = jnp.zeros((RPT, D), jnp.float32)

    deg_p = _deg_kernel(src, dst, ones_o, ones_i, zeros_h)

    x_pad = jnp.pad(features, ((0, N_PAD - N), (0, 0)))

    t0 = _tc_first(x_pad, W0, deg_p)
    a0 = _spmm_kernel(t0, src, dst, zeros_d)
    t1 = _tc_mid(a0, W1, b0.reshape(1, D), deg_p)
    a1 = _spmm_kernel(t1, src, dst, zeros_d)
    t2 = _tc_mid(a1, W2, b1.reshape(1, D), deg_p)
    a2 = _spmm_kernel(t2, src, dst, zeros_d)
    logits = _tc_last(a2, b2.reshape(1, D), deg_p)
    return logits[:N]


# speedup vs baseline: 11.0915x; 1.3858x over previous
import functools
import jax
import jax.numpy as jnp
from jax import lax
from jax.experimental import pallas as pl
from jax.experimental.pallas import tpu as pltpu
from jax.experimental.pallas import tpu_sc as plsc

N = 10000
E = 320000
D = 128

NC = 2
NS = 16
NW = NC * NS
EPW = E // NW
WIN = 128
NWIN = EPW // WIN
TAIL = EPW - NWIN * WIN

N_PAD = 10240
RPT = N_PAD // NS
HK = 16
DW = 128

_sc_mesh = plsc.VectorSubcoreMesh(core_axis_name="c", subcore_axis_name="s")


def _deg_body(src_hbm, dst_hbm, ones_o_hbm, ones_i_hbm, zeros_hbm,
              deg_hbm,
              idx_o, idx_i, idx_ot, idx_it, vals_o, vals_i, acc, sem):
    c = lax.axis_index("c")
    s = lax.axis_index("s")
    base = (c * NS + s) * EPW

    row0 = s * RPT
    pltpu.sync_copy(zeros_hbm, acc.at[pl.ds(row0, RPT)])
    pltpu.sync_copy(ones_o_hbm, vals_o)
    pltpu.sync_copy(ones_i_hbm, vals_i)
    plsc.subcore_barrier()

    @pl.loop(0, NWIN)
    def _win(g):
        off = pl.multiple_of(base + g * WIN, 16)
        pltpu.sync_copy(src_hbm.at[pl.ds(off, WIN)], idx_o)
        pltpu.sync_copy(dst_hbm.at[pl.ds(off, WIN)], idx_i)
        pltpu.sync_copy(vals_o, acc.at[idx_o], add=True)
        pltpu.sync_copy(vals_i, acc.at[idx_i], add=True)

    off_t = pl.multiple_of(base + NWIN * WIN, 16)
    pltpu.sync_copy(src_hbm.at[pl.ds(off_t, TAIL)], idx_ot)
    pltpu.sync_copy(dst_hbm.at[pl.ds(off_t, TAIL)], idx_it)
    pltpu.sync_copy(vals_o.at[pl.ds(0, TAIL)], acc.at[idx_ot], add=True)
    pltpu.sync_copy(vals_i.at[pl.ds(0, TAIL)], acc.at[idx_it], add=True)

    plsc.subcore_barrier()
    pltpu.sync_copy(acc.at[pl.ds(row0, RPT)], deg_hbm.at[c, pl.ds(row0, RPT)])


_deg_kernel = functools.partial(
    pl.kernel,
    out_type=jax.ShapeDtypeStruct((NC, N_PAD, DW), jnp.float32),
    mesh=_sc_mesh,
    scratch_types=[
        pltpu.VMEM((WIN,), jnp.int32),
        pltpu.VMEM((WIN,), jnp.int32),
        pltpu.VMEM((TAIL,), jnp.int32),
        pltpu.VMEM((TAIL,), jnp.int32),
        pltpu.VMEM((WIN, DW), jnp.float32),
        pltpu.VMEM((WIN, DW), jnp.float32),
        pltpu.VMEM_SHARED((N_PAD, DW), jnp.float32),
        pltpu.SemaphoreType.DMA,
    ],
)(_deg_body)


def _spmm_body(t_hbm, src_hbm, dst_hbm, zeros_hbm, out_hbm,
               sidx0, didx0, sidx1, didx1, sidx_t, didx_t,
               rows0, rows1, rows_t, acc, sem0, sem1):
    c = lax.axis_index("c")
    s = lax.axis_index("s")
    base = (c * NS + s) * EPW

    row0 = s * RPT
    pltpu.sync_copy(zeros_hbm, acc.at[pl.ds(row0, RPT)])
    plsc.subcore_barrier()

    def issue(g, sidx, didx, rows, sem):
        off = pl.multiple_of(base + g * WIN, 16)
        pltpu.sync_copy(src_hbm.at[pl.ds(off, WIN)], sidx)
        pltpu.sync_copy(dst_hbm.at[pl.ds(off, WIN)], didx)
        pltpu.async_copy(t_hbm.at[sidx], rows, sem)

    def drain(sidx, didx, rows, sem):
        pltpu.make_async_copy(t_hbm.at[sidx], rows, sem).wait()
        pltpu.sync_copy(rows, acc.at[didx], add=True)

    issue(0, sidx0, didx0, rows0, sem0)

    @pl.loop(0, NWIN // 2)
    def _k(k):
        g = k * 2
        issue(g + 1, sidx1, didx1, rows1, sem1)
        drain(sidx0, didx0, rows0, sem0)

        @pl.when(g + 2 < NWIN)
        def _():
            issue(g + 2, sidx0, didx0, rows0, sem0)

        drain(sidx1, didx1, rows1, sem1)

    off_t = pl.multiple_of(base + NWIN * WIN, 16)
    pltpu.sync_copy(src_hbm.at[pl.ds(off_t, TAIL)], sidx_t)
    pltpu.sync_copy(dst_hbm.at[pl.ds(off_t, TAIL)], didx_t)
    pltpu.async_copy(t_hbm.at[sidx_t], rows_t, sem0).wait()
    pltpu.sync_copy(rows_t, acc.at[didx_t], add=True)

    plsc.subcore_barrier()
    pltpu.sync_copy(acc.at[pl.ds(row0, RPT)], out_hbm.at[c, pl.ds(row0, RPT)])


_spmm_kernel = functools.partial(
    pl.kernel,
    out_type=jax.ShapeDtypeStruct((NC, N_PAD, D), jnp.float32),
    mesh=_sc_mesh,
    scratch_types=[
        pltpu.VMEM((WIN,), jnp.int32),
        pltpu.VMEM((WIN,), jnp.int32),
        pltpu.VMEM((WIN,), jnp.int32),
        pltpu.VMEM((WIN,), jnp.int32),
        pltpu.VMEM((TAIL,), jnp.int32),
        pltpu.VMEM((TAIL,), jnp.int32),
        pltpu.VMEM((WIN, D), jnp.float32),
        pltpu.VMEM((WIN, D), jnp.float32),
        pltpu.VMEM((TAIL, D), jnp.float32),
        pltpu.VMEM_SHARED((N_PAD, D), jnp.float32),
        pltpu.SemaphoreType.DMA,
        pltpu.SemaphoreType.DMA,
    ],
)(_spmm_body)


BLK = 1024
GRID = N_PAD // BLK


def _inv_sqrt_deg(deg_p_ref, i, col):
    deg = (deg_p_ref[0, pl.ds(i * BLK, BLK), pl.ds(col, HK)]
           + deg_p_ref[1, pl.ds(i * BLK, BLK), pl.ds(col, HK)])
    inv = jnp.where(deg > 0, lax.rsqrt(jnp.maximum(deg, 1.0)), 0.0)
    return inv[:, 0:1]


def _tc_first_body(x_ref, w_ref, deg_ref, out_ref):
    i = pl.program_id(0)
    inv_o = _inv_sqrt_deg(deg_ref, i, 0)
    t = jnp.dot(x_ref[...], w_ref[...], preferred_element_type=jnp.float32)
    out_ref[...] = t * inv_o


def _tc_mid_body(ap_ref, w_ref, b_ref, deg_ref, out_ref):
    i = pl.program_id(0)
    inv_i = _inv_sqrt_deg(deg_ref, i, HK)
    inv_o = _inv_sqrt_deg(deg_ref, i, 0)
    a = ap_ref[0] + ap_ref[1]
    h = jnp.maximum(a * inv_i + b_ref[...], 0.0)
    t = jnp.dot(h, w_ref[...], preferred_element_type=jnp.float32)
    out_ref[...] = t * inv_o


def _tc_last_body(ap_ref, b_ref, deg_ref, out_ref):
    i = pl.program_id(0)
    inv_i = _inv_sqrt_deg(deg_ref, i, HK)
    a = ap_ref[0] + ap_ref[1]
    out_ref[...] = a * inv_i + b_ref[...]


def _full(shape):
    return pl.BlockSpec(shape, lambda i: tuple(0 for _ in shape))


_row_spec = pl.BlockSpec((BLK, D), lambda i: (i, 0))
_ap_spec = pl.BlockSpec((NC, BLK, D), lambda i: (0, i, 0))
_deg_spec = _full((NC, N_PAD, DW))

_tc_first = pl.pallas_call(
    _tc_first_body,
    grid=(GRID,),
    in_specs=[_row_spec, _full((D, D)), _deg_spec],
    out_specs=_row_spec,
    out_shape=jax.ShapeDtypeStruct((N_PAD, D), jnp.float32),
)

_tc_mid = pl.pallas_call(
    _tc_mid_body,
    grid=(GRID,),
    in_specs=[_ap_spec, _full((D, D)), _full((1, D)), _deg_spec],
    out_specs=_row_spec,
    out_shape=jax.ShapeDtypeStruct((N_PAD, D), jnp.float32),
)

_tc_last = pl.pallas_call(
    _tc_last_body,
    grid=(GRID,),
    in_specs=[_ap_spec, _full((1, D)), _deg_spec],
    out_specs=_row_spec,
    out_shape=jax.ShapeDtypeStruct((N_PAD, D), jnp.float32),
)


@jax.jit
def kernel(features, edge_index, W0, b0, W1, b1, W2, b2):
    src = edge_index[0]
    dst = edge_index[1]

    col = jnp.arange(DW)
    ones_o = jnp.where(col < HK, 1.0, 0.0).astype(jnp.float32) * jnp.ones((WIN, 1), jnp.float32)
    ones_i = jnp.where((col >= HK) & (col < 2 * HK), 1.0, 0.0).astype(jnp.float32) * jnp.ones((WIN, 1), jnp.float32)
    zeros_h = jnp.zeros((RPT, DW), jnp.float32)
    zeros_d = jnp.zeros((RPT, D), jnp.float32)

    deg_p = _deg_kernel(src, dst, ones_o, ones_i, zeros_h)

    x_pad = jnp.pad(features, ((0, N_PAD - N), (0, 0)))

    t0 = _tc_first(x_pad, W0, deg_p)
    a0 = _spmm_kernel(t0, src, dst, zeros_d)
    t1 = _tc_mid(a0, W1, b0.reshape(1, D), deg_p)
    a1 = _spmm_kernel(t1, src, dst, zeros_d)
    t2 = _tc_mid(a1, W2, b1.reshape(1, D), deg_p)
    a2 = _spmm_kernel(t2, src, dst, zeros_d)
    logits = _tc_last(a2, b2.reshape(1, D), deg_p)
    return logits[:N]


# speedup vs baseline: 12.7921x; 1.1533x over previous
import functools
import jax
import jax.numpy as jnp
from jax import lax
from jax.experimental import pallas as pl
from jax.experimental.pallas import tpu as pltpu
from jax.experimental.pallas import tpu_sc as plsc

N = 10000
E = 320000
D = 128

NC = 2
NS = 16
NW = NC * NS
EPW = E // NW
WIN = 128
NWIN = EPW // WIN
TAIL = EPW - NWIN * WIN

N_PAD = 10240
RPT = N_PAD // NS
HK = 16
DW = 128

_sc_mesh = plsc.VectorSubcoreMesh(core_axis_name="c", subcore_axis_name="s")


def _deg_body(src_hbm, dst_hbm, ones_hbm, zeros_hbm, deg_hbm,
              idx_o, idx_i, idx_ot, idx_it, ones_v, acc_o, acc_i, sem):
    c = lax.axis_index("c")
    s = lax.axis_index("s")
    base = (c * NS + s) * EPW

    row0 = s * RPT
    pltpu.sync_copy(zeros_hbm, acc_o.at[pl.ds(row0, RPT)])
    pltpu.sync_copy(zeros_hbm, acc_i.at[pl.ds(row0, RPT)])
    pltpu.sync_copy(ones_hbm, ones_v)
    plsc.subcore_barrier()

    @pl.loop(0, NWIN)
    def _win(g):
        off = pl.multiple_of(base + g * WIN, 16)
        pltpu.sync_copy(src_hbm.at[pl.ds(off, WIN)], idx_o)
        pltpu.sync_copy(dst_hbm.at[pl.ds(off, WIN)], idx_i)
        pltpu.sync_copy(ones_v, acc_o.at[idx_o], add=True)
        pltpu.sync_copy(ones_v, acc_i.at[idx_i], add=True)

    off_t = pl.multiple_of(base + NWIN * WIN, 16)
    pltpu.sync_copy(src_hbm.at[pl.ds(off_t, TAIL)], idx_ot)
    pltpu.sync_copy(dst_hbm.at[pl.ds(off_t, TAIL)], idx_it)
    pltpu.sync_copy(ones_v.at[pl.ds(0, TAIL)], acc_o.at[idx_ot], add=True)
    pltpu.sync_copy(ones_v.at[pl.ds(0, TAIL)], acc_i.at[idx_it], add=True)

    plsc.subcore_barrier()
    pltpu.sync_copy(acc_o.at[pl.ds(row0, RPT)], deg_hbm.at[0, c, pl.ds(row0, RPT)])
    pltpu.sync_copy(acc_i.at[pl.ds(row0, RPT)], deg_hbm.at[1, c, pl.ds(row0, RPT)])


_deg_kernel = functools.partial(
    pl.kernel,
    out_type=jax.ShapeDtypeStruct((2, NC, N_PAD), jnp.float32),
    mesh=_sc_mesh,
    scratch_types=[
        pltpu.VMEM((WIN,), jnp.int32),
        pltpu.VMEM((WIN,), jnp.int32),
        pltpu.VMEM((TAIL,), jnp.int32),
        pltpu.VMEM((TAIL,), jnp.int32),
        pltpu.VMEM((WIN,), jnp.float32),
        pltpu.VMEM_SHARED((N_PAD,), jnp.float32),
        pltpu.VMEM_SHARED((N_PAD,), jnp.float32),
        pltpu.SemaphoreType.DMA,
    ],
)(_deg_body)


def _spmm_body(t_hbm, src_hbm, dst_hbm, zeros_hbm, out_hbm,
               sidx0, didx0, sidx1, didx1, sidx_t, didx_t,
               rows0, rows1, rows_t, acc, sem0, sem1):
    c = lax.axis_index("c")
    s = lax.axis_index("s")
    base = (c * NS + s) * EPW

    row0 = s * RPT
    pltpu.sync_copy(zeros_hbm, acc.at[pl.ds(row0, RPT)])
    plsc.subcore_barrier()

    def issue(g, sidx, didx, rows, sem):
        off = pl.multiple_of(base + g * WIN, 16)
        pltpu.sync_copy(src_hbm.at[pl.ds(off, WIN)], sidx)
        pltpu.sync_copy(dst_hbm.at[pl.ds(off, WIN)], didx)
        pltpu.async_copy(t_hbm.at[sidx], rows, sem)

    def drain(sidx, didx, rows, sem):
        pltpu.make_async_copy(t_hbm.at[sidx], rows, sem).wait()
        pltpu.sync_copy(rows, acc.at[didx], add=True)

    issue(0, sidx0, didx0, rows0, sem0)

    @pl.loop(0, NWIN // 2)
    def _k(k):
        g = k * 2
        issue(g + 1, sidx1, didx1, rows1, sem1)
        drain(sidx0, didx0, rows0, sem0)

        @pl.when(g + 2 < NWIN)
        def _():
            issue(g + 2, sidx0, didx0, rows0, sem0)

        drain(sidx1, didx1, rows1, sem1)

    off_t = pl.multiple_of(base + NWIN * WIN, 16)
    pltpu.sync_copy(src_hbm.at[pl.ds(off_t, TAIL)], sidx_t)
    pltpu.sync_copy(dst_hbm.at[pl.ds(off_t, TAIL)], didx_t)
    pltpu.async_copy(t_hbm.at[sidx_t], rows_t, sem0).wait()
    pltpu.sync_copy(rows_t, acc.at[didx_t], add=True)

    plsc.subcore_barrier()
    pltpu.sync_copy(acc.at[pl.ds(row0, RPT)], out_hbm.at[c, pl.ds(row0, RPT)])


_spmm_kernel = functools.partial(
    pl.kernel,
    out_type=jax.ShapeDtypeStruct((NC, N_PAD, D), jnp.float32),
    mesh=_sc_mesh,
    scratch_types=[
        pltpu.VMEM((WIN,), jnp.int32),
        pltpu.VMEM((WIN,), jnp.int32),
        pltpu.VMEM((WIN,), jnp.int32),
        pltpu.VMEM((WIN,), jnp.int32),
        pltpu.VMEM((TAIL,), jnp.int32),
        pltpu.VMEM((TAIL,), jnp.int32),
        pltpu.VMEM((WIN, D), jnp.float32),
        pltpu.VMEM((WIN, D), jnp.float32),
        pltpu.VMEM((TAIL, D), jnp.float32),
        pltpu.VMEM_SHARED((N_PAD, D), jnp.float32),
        pltpu.SemaphoreType.DMA,
        pltpu.SemaphoreType.DMA,
    ],
)(_spmm_body)


BLK = 1024
GRID = N_PAD // BLK


def _inv_sqrt_deg(deg_ref, i, col):
    blk = pl.ds(i * BLK, BLK)
    deg = deg_ref[blk, col:col + 1] + deg_ref[blk, col + 2:col + 3]
    return jnp.where(deg > 0, lax.rsqrt(jnp.maximum(deg, 1.0)), 0.0)


def _tc_first_body(x_ref, w_ref, deg_ref, out_ref):
    i = pl.program_id(0)
    inv_o = _inv_sqrt_deg(deg_ref, i, 0)
    t = jnp.dot(x_ref[...], w_ref[...], preferred_element_type=jnp.float32)
    out_ref[...] = t * inv_o


def _tc_mid_body(ap_ref, w_ref, b_ref, deg_ref, out_ref):
    i = pl.program_id(0)
    inv_i = _inv_sqrt_deg(deg_ref, i, 1)
    inv_o = _inv_sqrt_deg(deg_ref, i, 0)
    a = ap_ref[0] + ap_ref[1]
    h = jnp.maximum(a * inv_i + b_ref[...], 0.0)
    t = jnp.dot(h, w_ref[...], preferred_element_type=jnp.float32)
    out_ref[...] = t * inv_o


def _tc_last_body(ap_ref, b_ref, deg_ref, out_ref):
    i = pl.program_id(0)
    inv_i = _inv_sqrt_deg(deg_ref, i, 1)
    a = ap_ref[0] + ap_ref[1]
    out_ref[...] = a * inv_i + b_ref[...]


def _full(shape):
    return pl.BlockSpec(shape, lambda i: tuple(0 for _ in shape))


_row_spec = pl.BlockSpec((BLK, D), lambda i: (i, 0))
_ap_spec = pl.BlockSpec((NC, BLK, D), lambda i: (0, i, 0))
_deg_spec = _full((N_PAD, NC * 2))

_tc_first = pl.pallas_call(
    _tc_first_body,
    grid=(GRID,),
    in_specs=[_row_spec, _full((D, D)), _deg_spec],
    out_specs=_row_spec,
    out_shape=jax.ShapeDtypeStruct((N_PAD, D), jnp.float32),
)

_tc_mid = pl.pallas_call(
    _tc_mid_body,
    grid=(GRID,),
    in_specs=[_ap_spec, _full((D, D)), _full((1, D)), _deg_spec],
    out_specs=_row_spec,
    out_shape=jax.ShapeDtypeStruct((N_PAD, D), jnp.float32),
)

_tc_last = pl.pallas_call(
    _tc_last_body,
    grid=(GRID,),
    in_specs=[_ap_spec, _full((1, D)), _deg_spec],
    out_specs=_row_spec,
    out_shape=jax.ShapeDtypeStruct((N_PAD, D), jnp.float32),
)


@jax.jit
def kernel(features, edge_index, W0, b0, W1, b1, W2, b2):
    src = edge_index[0]
    dst = edge_index[1]

    ones_1 = jnp.ones((WIN,), jnp.float32)
    zeros_1 = jnp.zeros((RPT,), jnp.float32)
    zeros_d = jnp.zeros((RPT, D), jnp.float32)

    deg_raw = _deg_kernel(src, dst, ones_1, zeros_1)
    deg_p = jnp.transpose(deg_raw, (2, 1, 0)).reshape(N_PAD, NC * 2)

    x_pad = jnp.pad(features, ((0, N_PAD - N), (0, 0)))

    t0 = _tc_first(x_pad, W0, deg_p)
    a0 = _spmm_kernel(t0, src, dst, zeros_d)
    t1 = _tc_mid(a0, W1, b0.reshape(1, D), deg_p)
    a1 = _spmm_kernel(t1, src, dst, zeros_d)
    t2 = _tc_mid(a1, W2, b1.reshape(1, D), deg_p)
    a2 = _spmm_kernel(t2, src, dst, zeros_d)
    logits = _tc_last(a2, b2.reshape(1, D), deg_p)
    return logits[:N]


# speedup vs baseline: 12.9674x; 1.0137x over previous
import functools
import jax
import jax.numpy as jnp
from jax import lax
from jax.experimental import pallas as pl
from jax.experimental.pallas import tpu as pltpu
from jax.experimental.pallas import tpu_sc as plsc

N = 10000
E = 320000
D = 128

NC = 2
NS = 16
NW = NC * NS
EPW = E // NW
WIN = 128
NWIN = EPW // WIN
TAIL = EPW - NWIN * WIN
WIN_S = 104
NWIN_S = EPW // WIN_S
NWIN_S3 = NWIN_S // 3
TAIL_S = EPW - NWIN_S * WIN_S

N_PAD = 10240
RPT = N_PAD // NS
HK = 16
DW = 128

_sc_mesh = plsc.VectorSubcoreMesh(core_axis_name="c", subcore_axis_name="s")


def _deg_body(src_hbm, dst_hbm, ones_hbm, zeros_hbm, deg_hbm,
              idx_o, idx_i, idx_ob, idx_ib, idx_ot, idx_it, ones_v,
              acc_o, acc_i, so_a, si_a, so_b, si_b):
    c = lax.axis_index("c")
    s = lax.axis_index("s")
    base = (c * NS + s) * EPW

    row0 = s * RPT
    pltpu.sync_copy(zeros_hbm, acc_o.at[pl.ds(row0, RPT)])
    pltpu.sync_copy(zeros_hbm, acc_i.at[pl.ds(row0, RPT)])
    pltpu.sync_copy(ones_hbm, ones_v)
    plsc.subcore_barrier()

    @pl.loop(0, NWIN // 2)
    def _win(k):
        off_a = pl.multiple_of(base + (2 * k) * WIN, 16)
        pltpu.sync_copy(src_hbm.at[pl.ds(off_a, WIN)], idx_o)
        pltpu.sync_copy(dst_hbm.at[pl.ds(off_a, WIN)], idx_i)
        pltpu.async_copy(ones_v, acc_o.at[idx_o], so_a, add=True)
        pltpu.async_copy(ones_v, acc_i.at[idx_i], si_a, add=True)
        off_b = pl.multiple_of(base + (2 * k + 1) * WIN, 16)
        pltpu.sync_copy(src_hbm.at[pl.ds(off_b, WIN)], idx_ob)
        pltpu.sync_copy(dst_hbm.at[pl.ds(off_b, WIN)], idx_ib)
        pltpu.async_copy(ones_v, acc_o.at[idx_ob], so_b, add=True)
        pltpu.async_copy(ones_v, acc_i.at[idx_ib], si_b, add=True)
        pltpu.make_async_copy(ones_v, acc_o.at[idx_o], so_a).wait()
        pltpu.make_async_copy(ones_v, acc_i.at[idx_i], si_a).wait()
        pltpu.make_async_copy(ones_v, acc_o.at[idx_ob], so_b).wait()
        pltpu.make_async_copy(ones_v, acc_i.at[idx_ib], si_b).wait()

    off_t = pl.multiple_of(base + NWIN * WIN, 16)
    pltpu.sync_copy(src_hbm.at[pl.ds(off_t, TAIL)], idx_ot)
    pltpu.sync_copy(dst_hbm.at[pl.ds(off_t, TAIL)], idx_it)
    pltpu.sync_copy(ones_v.at[pl.ds(0, TAIL)], acc_o.at[idx_ot], add=True)
    pltpu.sync_copy(ones_v.at[pl.ds(0, TAIL)], acc_i.at[idx_it], add=True)

    plsc.subcore_barrier()
    pltpu.sync_copy(acc_o.at[pl.ds(row0, RPT)], deg_hbm.at[0, c, pl.ds(row0, RPT)])
    pltpu.sync_copy(acc_i.at[pl.ds(row0, RPT)], deg_hbm.at[1, c, pl.ds(row0, RPT)])


_deg_kernel = functools.partial(
    pl.kernel,
    out_type=jax.ShapeDtypeStruct((2, NC, N_PAD), jnp.float32),
    mesh=_sc_mesh,
    scratch_types=[
        pltpu.VMEM((WIN,), jnp.int32),
        pltpu.VMEM((WIN,), jnp.int32),
        pltpu.VMEM((WIN,), jnp.int32),
        pltpu.VMEM((WIN,), jnp.int32),
        pltpu.VMEM((TAIL,), jnp.int32),
        pltpu.VMEM((TAIL,), jnp.int32),
        pltpu.VMEM((WIN,), jnp.float32),
        pltpu.VMEM_SHARED((N_PAD,), jnp.float32),
        pltpu.VMEM_SHARED((N_PAD,), jnp.float32),
        pltpu.SemaphoreType.DMA,
        pltpu.SemaphoreType.DMA,
        pltpu.SemaphoreType.DMA,
        pltpu.SemaphoreType.DMA,
    ],
)(_deg_body)


def _spmm_body(t_hbm, src_hbm, dst_hbm, zeros_hbm, out_hbm,
               sidx0, didx0, sidx1, didx1, sidx2, didx2, didx_t,
               rows0, rows1, rows2, acc,
               gs0, gs1, gs2, ss0, ss1, ss2):
    c = lax.axis_index("c")
    s = lax.axis_index("s")
    base = (c * NS + s) * EPW

    row0 = s * RPT
    pltpu.sync_copy(zeros_hbm, acc.at[pl.ds(row0, RPT)])
    plsc.subcore_barrier()

    bufs = ((sidx0, didx0, rows0, gs0, ss0),
            (sidx1, didx1, rows1, gs1, ss1),
            (sidx2, didx2, rows2, gs2, ss2))

    def issue_gather(g, sidx, didx, rows, gs):
        off = pl.multiple_of(base + g * WIN_S, 8)
        pltpu.sync_copy(src_hbm.at[pl.ds(off, WIN_S)], sidx)
        pltpu.sync_copy(dst_hbm.at[pl.ds(off, WIN_S)], didx)
        pltpu.async_copy(t_hbm.at[sidx], rows, gs)

    for j in range(3):
        sidx, didx, rows, gs, ss = bufs[j]
        issue_gather(j, sidx, didx, rows, gs)

    @pl.loop(0, NWIN_S3)
    def _k(k):
        g = k * 3
        for j in range(3):
            sidx, didx, rows, gs, ss = bufs[j]
            pltpu.make_async_copy(t_hbm.at[sidx], rows, gs).wait()
            pltpu.async_copy(rows, acc.at[didx], ss, add=True)
        for j in range(3):
            sidx, didx, rows, gs, ss = bufs[j]

            @pl.when(g + 3 + j < NWIN_S)
            def _():
                pltpu.make_async_copy(rows, acc.at[didx], ss).wait()
                issue_gather(g + 3 + j, sidx, didx, rows, gs)

    for j in range(3):
        sidx, didx, rows, gs, ss = bufs[j]
        pltpu.make_async_copy(rows, acc.at[didx], ss).wait()

    off_t = pl.multiple_of(base + NWIN_S * WIN_S, 8)
    pltpu.sync_copy(src_hbm.at[pl.ds(off_t, TAIL_S)], sidx0.at[pl.ds(0, TAIL_S)])
    pltpu.sync_copy(dst_hbm.at[pl.ds(off_t, TAIL_S)], didx_t)
    rows_t = rows0.at[pl.ds(0, TAIL_S)]
    pltpu.async_copy(t_hbm.at[sidx0.at[pl.ds(0, TAIL_S)]], rows_t, gs0).wait()
    pltpu.sync_copy(rows_t, acc.at[didx_t], add=True)

    plsc.subcore_barrier()
    pltpu.sync_copy(acc.at[pl.ds(row0, RPT)], out_hbm.at[c, pl.ds(row0, RPT)])


_spmm_kernel = functools.partial(
    pl.kernel,
    out_type=jax.ShapeDtypeStruct((NC, N_PAD, D), jnp.float32),
    mesh=_sc_mesh,
    scratch_types=[
        pltpu.VMEM((WIN_S,), jnp.int32),
        pltpu.VMEM((WIN_S,), jnp.int32),
        pltpu.VMEM((WIN_S,), jnp.int32),
        pltpu.VMEM((WIN_S,), jnp.int32),
        pltpu.VMEM((WIN_S,), jnp.int32),
        pltpu.VMEM((WIN_S,), jnp.int32),
        pltpu.VMEM((TAIL_S,), jnp.int32),
        pltpu.VMEM((WIN_S, D), jnp.float32),
        pltpu.VMEM((WIN_S, D), jnp.float32),
        pltpu.VMEM((WIN_S, D), jnp.float32),
        pltpu.VMEM_SHARED((N_PAD, D), jnp.float32),
        pltpu.SemaphoreType.DMA,
        pltpu.SemaphoreType.DMA,
        pltpu.SemaphoreType.DMA,
        pltpu.SemaphoreType.DMA,
        pltpu.SemaphoreType.DMA,
        pltpu.SemaphoreType.DMA,
    ],
)(_spmm_body)


BLK = 1024
GRID = N_PAD // BLK


def _inv_sqrt_deg(deg_ref, i, col):
    blk = pl.ds(i * BLK, BLK)
    deg = deg_ref[blk, col:col + 1] + deg_ref[blk, col + 2:col + 3]
    return jnp.where(deg > 0, lax.rsqrt(jnp.maximum(deg, 1.0)), 0.0)


def _tc_first_body(x_ref, w_ref, deg_ref, out_ref):
    i = pl.program_id(0)
    inv_o = _inv_sqrt_deg(deg_ref, i, 0)
    t = jnp.dot(x_ref[...], w_ref[...], preferred_element_type=jnp.float32)
    out_ref[...] = t * inv_o


def _tc_mid_body(ap_ref, w_ref, b_ref, deg_ref, out_ref):
    i = pl.program_id(0)
    inv_i = _inv_sqrt_deg(deg_ref, i, 1)
    inv_o = _inv_sqrt_deg(deg_ref, i, 0)
    a = ap_ref[0] + ap_ref[1]
    h = jnp.maximum(a * inv_i + b_ref[...], 0.0)
    t = jnp.dot(h, w_ref[...], preferred_element_type=jnp.float32)
    out_ref[...] = t * inv_o


def _tc_last_body(ap_ref, b_ref, deg_ref, out_ref):
    i = pl.program_id(0)
    inv_i = _inv_sqrt_deg(deg_ref, i, 1)
    a = ap_ref[0] + ap_ref[1]
    out_ref[...] = a * inv_i + b_ref[...]


def _full(shape):
    return pl.BlockSpec(shape, lambda i: tuple(0 for _ in shape))


_row_spec = pl.BlockSpec((BLK, D), lambda i: (i, 0))
_ap_spec = pl.BlockSpec((NC, BLK, D), lambda i: (0, i, 0))
_deg_spec = _full((N_PAD, NC * 2))

_tc_first = pl.pallas_call(
    _tc_first_body,
    grid=(GRID,),
    in_specs=[_row_spec, _full((D, D)), _deg_spec],
    out_specs=_row_spec,
    out_shape=jax.ShapeDtypeStruct((N_PAD, D), jnp.float32),
)

_tc_mid = pl.pallas_call(
    _tc_mid_body,
    grid=(GRID,),
    in_specs=[_ap_spec, _full((D, D)), _full((1, D)), _deg_spec],
    out_specs=_row_spec,
    out_shape=jax.ShapeDtypeStruct((N_PAD, D), jnp.float32),
)

_tc_last = pl.pallas_call(
    _tc_last_body,
    grid=(GRID,),
    in_specs=[_ap_spec, _full((1, D)), _deg_spec],
    out_specs=_row_spec,
    out_shape=jax.ShapeDtypeStruct((N_PAD, D), jnp.float32),
)


@jax.jit
def kernel(features, edge_index, W0, b0, W1, b1, W2, b2):
    src = edge_index[0]
    dst = edge_index[1]

    ones_1 = jnp.ones((WIN,), jnp.float32)
    zeros_1 = jnp.zeros((RPT,), jnp.float32)
    zeros_d = jnp.zeros((RPT, D), jnp.float32)

    deg_raw = _deg_kernel(src, dst, ones_1, zeros_1)
    deg_p = jnp.transpose(deg_raw, (2, 1, 0)).reshape(N_PAD, NC * 2)

    x_pad = jnp.pad(features, ((0, N_PAD - N), (0, 0)))

    t0 = _tc_first(x_pad, W0, deg_p)
    a0 = _spmm_kernel(t0, src, dst, zeros_d)
    t1 = _tc_mid(a0, W1, b0.reshape(1, D), deg_p)
    a1 = _spmm_kernel(t1, src, dst, zeros_d)
    t2 = _tc_mid(a1, W2, b1.reshape(1, D), deg_p)
    a2 = _spmm_kernel(t2, src, dst, zeros_d)
    logits = _tc_last(a2, b2.reshape(1, D), deg_p)
    return logits[:N]
